# baseline (speedup 1.0000x reference)
import functools
import jax
import jax.numpy as jnp
from jax import lax
from jax.experimental import pallas as pl
from jax.experimental.pallas import tpu as pltpu

B = 16384
NUM_CLASSES = 101
OUT_DIM = 128
TPAD = 128
BLK = 512
NB = B // BLK


def _body(x_ref, w_ref, o_ref):
    xb = x_ref[0]
    iota = lax.broadcasted_iota(jnp.int32, (BLK, TPAD), 1)
    z = (xb == iota).astype(jnp.float32)
    y = jnp.dot(z, w_ref[...], preferred_element_type=jnp.float32)
    o_ref[...] = 1.0 / (1.0 + jnp.exp(-y))


@functools.partial(jax.jit, donate_argnums=())
def _run(x_r, w_pad):
    return pl.pallas_call(
        _body,
        grid=(NB,),
        in_specs=[
            pl.BlockSpec((1, BLK, 1), lambda i: (i, 0, 0)),
            pl.BlockSpec((TPAD, OUT_DIM), lambda i: (0, 0)),
        ],
        out_specs=pl.BlockSpec((BLK, OUT_DIM), lambda i: (i, 0)),
        out_shape=jax.ShapeDtypeStruct((B, OUT_DIM), jnp.float32),
    )(x_r, w_pad)


def kernel(x, W):
    x_r = x.reshape(NB, BLK, 1)
    w_pad = jnp.zeros((TPAD, OUT_DIM), jnp.float32).at[:NUM_CLASSES].set(W.T)
    return _run(x_r, w_pad)


# speedup vs baseline: 1.6871x; 1.6871x over previous
import functools
import jax
import jax.numpy as jnp
from jax import lax
from jax.experimental import pallas as pl
from jax.experimental.pallas import tpu as pltpu

B = 16384
NUM_CLASSES = 101
OUT_DIM = 128
TPAD = 128
BLK = 512
NB = B // BLK


def _body(x_ref, w_ref, o_ref):
    xb = x_ref[0]
    iota = lax.broadcasted_iota(jnp.int32, (TPAD, BLK), 0)
    zt = (xb == iota).astype(jnp.float32)
    y = lax.dot_general(
        zt, w_ref[...],
        dimension_numbers=(((0,), (0,)), ((), ())),
        preferred_element_type=jnp.float32,
    )
    o_ref[...] = 1.0 / (1.0 + jnp.exp(-y))


@jax.jit
def _run(x_r, w_pad):
    return pl.pallas_call(
        _body,
        grid=(NB,),
        in_specs=[
            pl.BlockSpec((1, 1, BLK), lambda i: (i, 0, 0)),
            pl.BlockSpec((TPAD, OUT_DIM), lambda i: (0, 0)),
        ],
        out_specs=pl.BlockSpec((BLK, OUT_DIM), lambda i: (i, 0)),
        out_shape=jax.ShapeDtypeStruct((B, OUT_DIM), jnp.float32),
    )(x_r, w_pad)


def kernel(x, W):
    x_r = x.reshape(NB, 1, BLK)
    w_pad = jnp.zeros((TPAD, OUT_DIM), jnp.float32).at[:NUM_CLASSES].set(W.T)
    return _run(x_r, w_pad)


# speedup vs baseline: 4.0126x; 2.3783x over previous
import jax
import jax.numpy as jnp
from jax import lax
from jax.experimental import pallas as pl
from jax.experimental.pallas import tpu as pltpu

B = 16384
NUM_CLASSES = 101
OUT_DIM = 128
TPAD = 128
BLK = 2048
NB = B // BLK


def _body(x_ref, w_ref, o_ref):
    xb = x_ref[0]
    iota = lax.broadcasted_iota(jnp.int32, (TPAD, BLK), 0)
    zt = (xb == iota).astype(jnp.float32)
    w = w_ref[...]
    sig = 1.0 / (1.0 + jnp.exp(-w))
    sig = jnp.concatenate(
        [sig, jnp.zeros((OUT_DIM, TPAD - NUM_CLASSES), jnp.float32)], axis=1
    )
    tbl = jnp.transpose(sig)
    o_ref[...] = lax.dot_general(
        zt, tbl,
        dimension_numbers=(((0,), (0,)), ((), ())),
        preferred_element_type=jnp.float32,
    )


@jax.jit
def _run(x_r, w):
    return pl.pallas_call(
        _body,
        grid=(NB,),
        in_specs=[
            pl.BlockSpec((1, 1, BLK), lambda i: (i, 0, 0)),
            pl.BlockSpec((OUT_DIM, NUM_CLASSES), lambda i: (0, 0)),
        ],
        out_specs=pl.BlockSpec((BLK, OUT_DIM), lambda i: (i, 0)),
        out_shape=jax.ShapeDtypeStruct((B, OUT_DIM), jnp.float32),
    )(x_r, w)


def kernel(x, W):
    return _run(x.reshape(NB, 1, BLK), W)
